# baseline (speedup 1.0000x reference)
import functools

import jax
import jax.numpy as jnp
from jax import lax
from jax.experimental import pallas as pl
from jax.experimental.pallas import tpu as pltpu
from jax.experimental.pallas import tpu_sc as plsc

N_NODES = 10000
N_PAD = 10240
NC, NS, LANES = 2, 16, 16
NW = NC * NS
CHUNK = 128
RPT = N_PAD // NS
BLK = 1000


def _sc_mesh():
    return plsc.VectorSubcoreMesh(
        core_axis_name="c", subcore_axis_name="s", num_cores=NC, num_subcores=NS)


_SC_PARAMS = pltpu.CompilerParams(use_tc_tiling_on_sc=False)


@functools.lru_cache(maxsize=None)
def _make_deg(nm, ntw):
    @functools.partial(
        pl.kernel,
        out_type=jax.ShapeDtypeStruct((NC, N_PAD), jnp.float32),
        mesh=_sc_mesh(),
        scratch_types=[
            pltpu.VMEM((nm + 1, CHUNK), jnp.int32),
            pltpu.VMEM((CHUNK,), jnp.float32),
            pltpu.VMEM((RPT,), jnp.float32),
            pltpu.VMEM_SHARED((N_PAD,), jnp.float32),
        ],
        compiler_params=_SC_PARAMS,
    )
    def deg_kernel(ei_hbm, out_hbm, dst_v, ones_v, zeros_v, acc):
        cid = lax.axis_index("c")
        sid = lax.axis_index("s")
        wid = sid * NC + cid
        one = jnp.ones((LANES,), jnp.float32)
        zro = jnp.zeros((LANES,), jnp.float32)
        for c in range(CHUNK // LANES):
            ones_v[pl.ds(c * LANES, LANES)] = one

        def zb(i, carry):
            zeros_v[pl.ds(i * LANES, LANES)] = zro
            return carry

        lax.fori_loop(0, RPT // LANES, zb, 0)
        pltpu.sync_copy(zeros_v, acc.at[pl.ds(sid * RPT, RPT)])
        pltpu.sync_copy(ei_hbm.at[1, pl.ds(wid * nm, nm)], dst_v.at[pl.ds(0, nm)])

        @pl.when(wid < ntw)
        def _():
            pltpu.sync_copy(ei_hbm.at[1, NW * nm + wid], dst_v.at[nm])

        plsc.subcore_barrier()
        nch_w = nm + jnp.where(wid < ntw, 1, 0)

        def body(j, carry):
            pltpu.sync_copy(ones_v, acc.at[dst_v.at[j]], add=True)
            return carry

        lax.fori_loop(0, nch_w, body, 0)
        plsc.subcore_barrier()
        pltpu.sync_copy(acc.at[pl.ds(sid * RPT, RPT)],
                        out_hbm.at[cid, pl.ds(sid * RPT, RPT)])

    return deg_kernel


@functools.lru_cache(maxsize=None)
def _make_agg(nm, ntw, feat):
    NB = 8
    assert nm >= NB and nm % NB == NB - 1, "ring epilogue covers nm%NB+1 chunks"
    ngroups = nm // NB

    @functools.partial(
        pl.kernel,
        out_type=jax.ShapeDtypeStruct((NC, N_PAD, feat), jnp.float32),
        mesh=_sc_mesh(),
        scratch_types=[
            pltpu.VMEM((nm + 1, CHUNK), jnp.int32),
            pltpu.VMEM((nm + 1, CHUNK), jnp.int32),
            pltpu.VMEM((NB * CHUNK, feat), jnp.float32),
            pltpu.VMEM((RPT, feat), jnp.float32),
            pltpu.VMEM_SHARED((N_PAD, feat), jnp.float32),
            [pltpu.SemaphoreType.DMA] * NB,
            [pltpu.SemaphoreType.DMA] * NB,
        ],
        compiler_params=_SC_PARAMS,
    )
    def agg_kernel(g_hbm, ei_hbm, out_hbm,
                   src_v, dst_v, rows, zst, acc, gsem, ssem):
        cid = lax.axis_index("c")
        sid = lax.axis_index("s")
        wid = sid * NC + cid
        zro = jnp.zeros((LANES,), jnp.float32)

        def zb(r, carry):
            for c in range(feat // LANES):
                zst[r, pl.ds(c * LANES, LANES)] = zro
            return carry

        lax.fori_loop(0, RPT, zb, 0)
        pltpu.sync_copy(zst, acc.at[pl.ds(sid * RPT, RPT)])
        pltpu.sync_copy(ei_hbm.at[0, pl.ds(wid * nm, nm)], src_v.at[pl.ds(0, nm)])
        pltpu.sync_copy(ei_hbm.at[1, pl.ds(wid * nm, nm)], dst_v.at[pl.ds(0, nm)])
        has_tail = wid < ntw

        @pl.when(has_tail)
        def _():
            pltpu.sync_copy(ei_hbm.at[0, NW * nm + wid], src_v.at[nm])
            pltpu.sync_copy(ei_hbm.at[1, NW * nm + wid], dst_v.at[nm])

        plsc.subcore_barrier()
        n_total = nm + jnp.where(has_tail, 1, 0)

        def buf(b):
            return rows.at[pl.ds(b * CHUNK, CHUNK)]

        def gather_start(j, b):
            pltpu.async_copy(g_hbm.at[src_v.at[j]], buf(b), gsem[b])

        def gather_wait(j, b):
            pltpu.make_async_copy(g_hbm.at[src_v.at[j]], buf(b), gsem[b]).wait()

        def scatter_start(j, b):
            pltpu.async_copy(buf(b), acc.at[dst_v.at[j]], ssem[b], add=True)

        def scatter_wait(j, b):
            pltpu.make_async_copy(buf(b), acc.at[dst_v.at[j]], ssem[b]).wait()

        for b in range(NB):
            gather_start(b, b)

        def body(i, carry):
            j0 = NB * i
            for b in range(NB):
                gather_wait(j0 + b, b)
                scatter_start(j0 + b, b)
            for b in range(NB):
                k = j0 + NB + b
                scatter_wait(j0 + b, b)

                @pl.when(k < n_total)
                def _():
                    gather_start(k, b)

            return carry

        lax.fori_loop(0, ngroups, body, 0)
        j0 = NB * ngroups
        for b in range(NB):
            k = j0 + b

            @pl.when(k < n_total)
            def _():
                gather_wait(k, b)
                scatter_start(k, b)

            @pl.when(k < n_total)
            def _():
                scatter_wait(k, b)

        plsc.subcore_barrier()
        pltpu.sync_copy(acc.at[pl.ds(sid * RPT, RPT)],
                        out_hbm.at[cid, pl.ds(sid * RPT, RPT)])

    return agg_kernel


def _dis_from(degp_ref):
    deg = degp_ref[0] + degp_ref[1] + 1.0
    return lax.rsqrt(deg)[:, None]


def _tc_first_body(x_ref, w1_ref, degp_ref, g1_ref, dis_ref):
    h = jnp.dot(x_ref[...], w1_ref[...], preferred_element_type=jnp.float32)
    dis = _dis_from(degp_ref)
    dis_ref[...] = dis
    g1_ref[...] = h * dis


def _tc_mid_body(s_ref, g_ref, dis_ref, b_ref, w_ref, gn_ref):
    dis = dis_ref[...]
    out = jnp.maximum((s_ref[0] + s_ref[1] + g_ref[...]) * dis
                      + b_ref[...], 0.0)
    h = jnp.dot(out, w_ref[...], preferred_element_type=jnp.float32)
    gn_ref[...] = h * dis


def _tc_last_body(s_ref, g_ref, dis_ref, b_ref, y_ref):
    s = s_ref[0, :N_NODES] + s_ref[1, :N_NODES] + g_ref[:N_NODES]
    out = jnp.maximum(s * dis_ref[:N_NODES] + b_ref[...], 0.0)
    m = jnp.max(out, axis=-1, keepdims=True)
    lse = jnp.log(jnp.sum(jnp.exp(out - m), axis=-1, keepdims=True)) + m
    y_ref[...] = out - lse


def _tc_first(x_p, w1, degp):
    d_out = w1.shape[1]
    return pl.pallas_call(
        _tc_first_body,
        out_shape=(jax.ShapeDtypeStruct((N_PAD, d_out), jnp.float32),
                   jax.ShapeDtypeStruct((N_PAD, 1), jnp.float32)),
    )(x_p, w1, degp)


def _tc_mid(s, g, dis, b, w):
    d_out = w.shape[1]
    return pl.pallas_call(
        _tc_mid_body,
        out_shape=jax.ShapeDtypeStruct((N_PAD, d_out), jnp.float32),
    )(s, g, dis, b, w)


def _tc_last(s, g, dis, b):
    feat = b.shape[-1]
    return pl.pallas_call(
        _tc_last_body,
        out_shape=jax.ShapeDtypeStruct((N_NODES, feat), jnp.float32),
    )(s, g, dis, b)


def kernel(x, edge_index, W1, b1, W2, b2, W3, b3):
    n_edges = edge_index.shape[1]
    nm = n_edges // (NW * CHUNK)
    assert nm % 2 == 1, "pairs loop assumes an odd number of full chunks"
    main = NW * nm * CHUNK
    rem = n_edges - main
    assert rem % CHUNK == 0
    ntw = rem // CHUNK
    assert 0 < ntw <= NW
    ei = edge_index.astype(jnp.int32).reshape(2, n_edges // CHUNK, CHUNK)
    x_p = jnp.pad(x, ((0, N_PAD - N_NODES), (0, 0)))

    degp = _make_deg(nm, ntw)(ei)

    g1, dis = _tc_first(x_p, W1, degp)
    agg1 = _make_agg(nm, ntw, W1.shape[1])
    agg2 = _make_agg(nm, ntw, W2.shape[1])
    agg3 = _make_agg(nm, ntw, W3.shape[1])
    s1 = agg1(g1, ei)
    g2 = _tc_mid(s1, g1, dis, b1.reshape(1, -1), W2)
    s2 = agg2(g2, ei)
    g3 = _tc_mid(s2, g2, dis, b2.reshape(1, -1), W3)
    s3 = agg3(g3, ei)
    return _tc_last(s3, g3, dis, b3.reshape(1, -1))


# speedup vs baseline: 1.0323x; 1.0323x over previous
import functools

import jax
import jax.numpy as jnp
from jax import lax
from jax.experimental import pallas as pl
from jax.experimental.pallas import tpu as pltpu
from jax.experimental.pallas import tpu_sc as plsc

N_NODES = 10000
N_PAD = 10240
NC, NS, LANES = 2, 16, 16
NW = NC * NS
CHUNK = 128
RPT = N_PAD // NS


def _sc_mesh():
    return plsc.VectorSubcoreMesh(
        core_axis_name="c", subcore_axis_name="s", num_cores=NC, num_subcores=NS)


_SC_PARAMS = pltpu.CompilerParams(use_tc_tiling_on_sc=False)


@functools.lru_cache(maxsize=None)
def _make_deg(nm, ntw):
    @functools.partial(
        pl.kernel,
        out_type=jax.ShapeDtypeStruct((NC, N_PAD), jnp.float32),
        mesh=_sc_mesh(),
        scratch_types=[
            pltpu.VMEM((nm + 1, CHUNK), jnp.int32),
            pltpu.VMEM((CHUNK,), jnp.float32),
            pltpu.VMEM((RPT,), jnp.float32),
            pltpu.VMEM_SHARED((N_PAD,), jnp.float32),
        ],
        compiler_params=_SC_PARAMS,
    )
    def deg_kernel(ei_hbm, out_hbm, dst_v, ones_v, zeros_v, acc):
        cid = lax.axis_index("c")
        sid = lax.axis_index("s")
        wid = sid * NC + cid
        one = jnp.ones((LANES,), jnp.float32)
        zro = jnp.zeros((LANES,), jnp.float32)
        for c in range(CHUNK // LANES):
            ones_v[pl.ds(c * LANES, LANES)] = one

        def zb(i, carry):
            zeros_v[pl.ds(i * LANES, LANES)] = zro
            return carry

        lax.fori_loop(0, RPT // LANES, zb, 0)
        pltpu.sync_copy(zeros_v, acc.at[pl.ds(sid * RPT, RPT)])
        pltpu.sync_copy(ei_hbm.at[1, pl.ds(wid * nm, nm)], dst_v.at[pl.ds(0, nm)])

        @pl.when(wid < ntw)
        def _():
            pltpu.sync_copy(ei_hbm.at[1, NW * nm + wid], dst_v.at[nm])

        plsc.subcore_barrier()
        nch_w = nm + jnp.where(wid < ntw, 1, 0)

        def body(j, carry):
            pltpu.sync_copy(ones_v, acc.at[dst_v.at[j]], add=True)
            return carry

        lax.fori_loop(0, nch_w, body, 0)
        plsc.subcore_barrier()
        pltpu.sync_copy(acc.at[pl.ds(sid * RPT, RPT)],
                        out_hbm.at[cid, pl.ds(sid * RPT, RPT)])

    return deg_kernel


@functools.lru_cache(maxsize=None)
def _make_agg(nm, ntw, feat):
    NB = 8
    assert nm >= NB and nm % NB == NB - 1, "ring epilogue covers nm%NB+1 chunks"
    ngroups = nm // NB

    @functools.partial(
        pl.kernel,
        out_type=jax.ShapeDtypeStruct((NC, N_PAD, feat), jnp.float32),
        mesh=_sc_mesh(),
        scratch_types=[
            pltpu.VMEM((nm + 1, CHUNK), jnp.int32),
            pltpu.VMEM((nm + 1, CHUNK), jnp.int32),
            pltpu.VMEM((NB * CHUNK, feat), jnp.float32),
            pltpu.VMEM((RPT, feat), jnp.float32),
            pltpu.VMEM_SHARED((N_PAD, feat), jnp.float32),
            [pltpu.SemaphoreType.DMA] * NB,
            [pltpu.SemaphoreType.DMA] * NB,
        ],
        compiler_params=_SC_PARAMS,
    )
    def agg_kernel(g_hbm, ei_hbm, out_hbm,
                   src_v, dst_v, rows, zst, acc, gsem, ssem):
        cid = lax.axis_index("c")
        sid = lax.axis_index("s")
        wid = sid * NC + cid
        zro = jnp.zeros((LANES,), jnp.float32)

        def zb(r, carry):
            for c in range(feat // LANES):
                zst[r, pl.ds(c * LANES, LANES)] = zro
            return carry

        lax.fori_loop(0, RPT, zb, 0)
        pltpu.sync_copy(zst, acc.at[pl.ds(sid * RPT, RPT)])
        pltpu.sync_copy(ei_hbm.at[0, pl.ds(wid * nm, nm)], src_v.at[pl.ds(0, nm)])
        pltpu.sync_copy(ei_hbm.at[1, pl.ds(wid * nm, nm)], dst_v.at[pl.ds(0, nm)])
        has_tail = wid < ntw

        @pl.when(has_tail)
        def _():
            pltpu.sync_copy(ei_hbm.at[0, NW * nm + wid], src_v.at[nm])
            pltpu.sync_copy(ei_hbm.at[1, NW * nm + wid], dst_v.at[nm])

        plsc.subcore_barrier()
        n_total = nm + jnp.where(has_tail, 1, 0)

        def buf(b):
            return rows.at[pl.ds(b * CHUNK, CHUNK)]

        def gather_start(j, b):
            pltpu.async_copy(g_hbm.at[src_v.at[j]], buf(b), gsem[b])

        def gather_wait(j, b):
            pltpu.make_async_copy(g_hbm.at[src_v.at[j]], buf(b), gsem[b]).wait()

        def scatter_start(j, b):
            pltpu.async_copy(buf(b), acc.at[dst_v.at[j]], ssem[b], add=True)

        def scatter_wait(j, b):
            pltpu.make_async_copy(buf(b), acc.at[dst_v.at[j]], ssem[b]).wait()

        for b in range(NB):
            gather_start(b, b)

        def body(i, carry):
            j0 = NB * i
            for b in range(NB):
                gather_wait(j0 + b, b)
                scatter_start(j0 + b, b)
            for b in range(NB):
                k = j0 + NB + b
                scatter_wait(j0 + b, b)

                @pl.when(k < n_total)
                def _():
                    gather_start(k, b)

            return carry

        lax.fori_loop(0, ngroups, body, 0)
        j0 = NB * ngroups
        for b in range(NB):
            k = j0 + b

            @pl.when(k < n_total)
            def _():
                gather_wait(k, b)
                scatter_start(k, b)

            @pl.when(k < n_total)
            def _():
                scatter_wait(k, b)

        plsc.subcore_barrier()
        pltpu.sync_copy(acc.at[pl.ds(sid * RPT, RPT)],
                        out_hbm.at[cid, pl.ds(sid * RPT, RPT)])

    return agg_kernel


def _dis_from(degp_ref):
    deg = degp_ref[0] + degp_ref[1] + 1.0
    return lax.rsqrt(deg)[:, None]


def _tc_first_body(x_ref, w1_ref, degp_ref, g1_ref):
    h = jnp.dot(x_ref[...], w1_ref[...], preferred_element_type=jnp.float32)
    g1_ref[...] = h * _dis_from(degp_ref)


def _tc_mid_body(s_ref, g_ref, degp_ref, b_ref, w_ref, gn_ref):
    dis = _dis_from(degp_ref)
    out = jnp.maximum((s_ref[0] + s_ref[1] + g_ref[...]) * dis
                      + b_ref[...], 0.0)
    h = jnp.dot(out, w_ref[...], preferred_element_type=jnp.float32)
    gn_ref[...] = h * dis


def _tc_last_body(s_ref, g_ref, degp_ref, b_ref, y_ref):
    dis = _dis_from(degp_ref)[:N_NODES]
    s = s_ref[0, :N_NODES] + s_ref[1, :N_NODES] + g_ref[:N_NODES]
    out = jnp.maximum(s * dis + b_ref[...], 0.0)
    m = jnp.max(out, axis=-1, keepdims=True)
    lse = jnp.log(jnp.sum(jnp.exp(out - m), axis=-1, keepdims=True)) + m
    y_ref[...] = out - lse


def _tc_first(x_p, w1, degp):
    d_out = w1.shape[1]
    return pl.pallas_call(
        _tc_first_body,
        out_shape=jax.ShapeDtypeStruct((N_PAD, d_out), jnp.float32),
    )(x_p, w1, degp)


def _tc_mid(s, g, degp, b, w):
    d_out = w.shape[1]
    return pl.pallas_call(
        _tc_mid_body,
        out_shape=jax.ShapeDtypeStruct((N_PAD, d_out), jnp.float32),
    )(s, g, degp, b, w)


def _tc_last(s, g, degp, b):
    feat = b.shape[-1]
    return pl.pallas_call(
        _tc_last_body,
        out_shape=jax.ShapeDtypeStruct((N_NODES, feat), jnp.float32),
    )(s, g, degp, b)


def kernel(x, edge_index, W1, b1, W2, b2, W3, b3):
    n_edges = edge_index.shape[1]
    nm = n_edges // (NW * CHUNK)
    assert nm % 2 == 1, "pairs loop assumes an odd number of full chunks"
    main = NW * nm * CHUNK
    rem = n_edges - main
    assert rem % CHUNK == 0
    ntw = rem // CHUNK
    assert 0 < ntw <= NW
    ei = edge_index.astype(jnp.int32).reshape(2, n_edges // CHUNK, CHUNK)
    x_p = jnp.pad(x, ((0, N_PAD - N_NODES), (0, 0)))

    degp = _make_deg(nm, ntw)(ei)

    g1 = _tc_first(x_p, W1, degp)
    agg1 = _make_agg(nm, ntw, W1.shape[1])
    agg2 = _make_agg(nm, ntw, W2.shape[1])
    agg3 = _make_agg(nm, ntw, W3.shape[1])
    s1 = agg1(g1, ei)
    g2 = _tc_mid(s1, g1, degp, b1.reshape(1, -1), W2)
    s2 = agg2(g2, ei)
    g3 = _tc_mid(s2, g2, degp, b2.reshape(1, -1), W3)
    s3 = agg3(g3, ei)
    return _tc_last(s3, g3, degp, b3.reshape(1, -1))
